# baseline (speedup 1.0000x reference)
import jax
import jax.numpy as jnp
from jax import lax
from jax.experimental import pallas as pl
from jax.experimental.pallas import tpu as pltpu
from jax.experimental.pallas import tpu_sc as plsc

_B, _S, _H = 4, 8192, 768
_K = _S // 2
_ALPHA = 0.1
_BS = 1024
_NBLK = (_B * _S) // _BS
_RPB = _S // _BS


def _router_body(x_ref, w_ref, lg_ref, sc_ref):
    x = x_ref[...]
    w = w_ref[...]
    lg = lax.dot_general(w, x, (((1,), (1,)), ((), ())),
                         preferred_element_type=jnp.float32)
    lg_ref[...] = lg[:, None, :]
    sc_ref[...] = _ALPHA / (1.0 + jnp.exp(-lg[:, None, :]))


def _threshold_body(sc_ref, lg_ref, thr_ref, ntk_ref, zl_ref):
    s = sc_ref[...]
    l = lg_ref[...]
    chunks = [s[b * _RPB:(b + 1) * _RPB] for b in range(_B)]

    def body(_, carry):
        los, his = carry
        new_lo, new_hi = [], []
        for b in range(_B):
            mid = los[b] + lax.shift_right_logical(his[b] - los[b], 1)
            t = lax.bitcast_convert_type(mid, jnp.float32)
            cnt = jnp.sum((chunks[b] >= t).astype(jnp.int32))
            pred = cnt >= _K
            new_lo.append(jnp.where(pred, mid, los[b]))
            new_hi.append(jnp.where(pred, his[b], mid))
        return (tuple(new_lo), tuple(new_hi))

    lo0 = tuple(jnp.int32(0) for _ in range(_B))
    hi0 = tuple(jnp.int32(0x7F800000) for _ in range(_B))
    los, _ = lax.fori_loop(0, 31, body, (lo0, hi0))
    thrs, ntks, zs = [], [], []
    for b in range(_B):
        t = lax.bitcast_convert_type(los[b], jnp.float32)
        cnt_gt = jnp.sum((chunks[b] > t).astype(jnp.int32))
        thrs.append(jnp.full((1, 16), t, jnp.float32))
        ntks.append(jnp.full((1, 16), _K - cnt_gt, jnp.int32))
        lc = l[b * _RPB:(b + 1) * _RPB]
        mx = jnp.max(lc)
        z = jnp.log(jnp.sum(jnp.exp(lc - mx))) + mx
        zs.append(z * z)
    thr_ref[...] = jnp.concatenate(thrs, axis=0)
    ntk_ref[...] = jnp.concatenate(ntks, axis=0)
    zl_ref[...] = jnp.full((1, 1), (zs[0] + zs[1] + zs[2] + zs[3]) / float(_B),
                           jnp.float32)


def _sc_compact_body(sc_hbm, thr_hbm, ntk_hbm, sel_out,
                     s_v, idx_v, thr_v, ntk_v):
    b = lax.axis_index("s") * 2 + lax.axis_index("c")

    @pl.when(b < _B)
    def _():
        pltpu.sync_copy(sc_hbm.at[pl.ds(b * _S, _S)], s_v)
        pltpu.sync_copy(thr_hbm.at[pl.ds(b * 16, 16)], thr_v)
        pltpu.sync_copy(ntk_hbm.at[pl.ds(b * 16, 16)], ntk_v)
        thr = thr_v[...]
        ntk = ntk_v[...]
        zero_i = jnp.zeros((16,), jnp.int32)
        one_i = jnp.ones((16,), jnp.int32)

        def body(i, carry):
            eq_seen, off = carry
            sl = s_v[pl.ds(i * 16, 16)]
            gt = sl > thr
            eq = sl == thr
            eqi = jnp.where(eq, one_i, zero_i)
            csum = plsc.cumsum(eqi)
            eq_seen_v = jnp.full((16,), eq_seen, jnp.int32)
            rank = (csum - eqi) + eq_seen_v
            take = jnp.logical_and(eq, rank < ntk)
            mask = jnp.logical_or(gt, take)
            idx = lax.iota(jnp.int32, 16) + jnp.full((16,), i * 16, jnp.int32)
            plsc.store_compressed(idx_v.at[pl.ds(off, 16)], idx, mask=mask)
            nsel = jnp.sum(jnp.where(mask, one_i, zero_i))
            neq = jnp.sum(eqi)
            return (eq_seen + neq, off + nsel)

        lax.fori_loop(0, _S // 16, body,
                      (jnp.int32(0), jnp.int32(0)), unroll=False)
        pltpu.sync_copy(idx_v.at[pl.ds(0, _K)], sel_out.at[pl.ds(b * _K, _K)])


def _update_body(x_ref, wr_ref, thr_ref, w_ref, o_ref):
    x = x_ref[...]
    wr = wr_ref[...]
    lg = lax.dot_general(x, wr, (((1,), (1,)), ((), ())),
                         preferred_element_type=jnp.float32)
    sc = _ALPHA / (1.0 + jnp.exp(-lg))
    thr = thr_ref[0, 0, 0]
    m = jnp.where(sc >= thr, sc, 0.0)
    y = jnp.dot(x.astype(jnp.bfloat16), w_ref[...],
                preferred_element_type=jnp.float32)
    o_ref[...] = x + m * y


def kernel(hidden_states, router_w, block_w):
    f32 = jnp.float32
    i32 = jnp.int32
    hs2d = hidden_states.reshape(_B * _S, _H)
    w_row = router_w.reshape(1, _H)

    lg_rows, sc_rows = pl.pallas_call(
        _router_body,
        grid=(_NBLK,),
        in_specs=[
            pl.BlockSpec((_BS, _H), lambda i: (i, 0)),
            pl.BlockSpec((1, _H), lambda i: (0, 0)),
        ],
        out_specs=[
            pl.BlockSpec((1, 1, _BS), lambda i: (i, 0, 0)),
            pl.BlockSpec((1, 1, _BS), lambda i: (i, 0, 0)),
        ],
        out_shape=[
            jax.ShapeDtypeStruct((_NBLK, 1, _BS), f32),
            jax.ShapeDtypeStruct((_NBLK, 1, _BS), f32),
        ],
    )(hs2d, w_row)

    thr16, ntk16, zl = pl.pallas_call(
        _threshold_body,
        out_shape=[
            jax.ShapeDtypeStruct((_B, 16), f32),
            jax.ShapeDtypeStruct((_B, 16), i32),
            jax.ShapeDtypeStruct((1, 1), f32),
        ],
    )(sc_rows, lg_rows)

    mesh = plsc.VectorSubcoreMesh(core_axis_name="c", subcore_axis_name="s")
    sel1d = pl.kernel(
        _sc_compact_body,
        out_type=jax.ShapeDtypeStruct((_B * _K,), i32),
        mesh=mesh,
        compiler_params=pltpu.CompilerParams(needs_layout_passes=False),
        scratch_types=[
            pltpu.VMEM((_S,), f32),
            pltpu.VMEM((_K + 16,), i32),
            pltpu.VMEM((16,), f32),
            pltpu.VMEM((16,), i32),
        ],
    )(sc_rows.reshape(_B * _S), thr16.reshape(_B * 16), ntk16.reshape(_B * 16))

    thr3d = thr16.reshape(_B, 1, 16)
    w_bf = block_w.astype(jnp.bfloat16)

    out2d = pl.pallas_call(
        _update_body,
        grid=(_NBLK,),
        in_specs=[
            pl.BlockSpec((_BS, _H), lambda i: (i, 0)),
            pl.BlockSpec((1, _H), lambda i: (0, 0)),
            pl.BlockSpec((1, 1, 16), lambda i: (i // _RPB, 0, 0)),
            pl.BlockSpec((_H, _H), lambda i: (0, 0)),
        ],
        out_specs=pl.BlockSpec((_BS, _H), lambda i: (i, 0)),
        out_shape=jax.ShapeDtypeStruct((_B * _S, _H), f32),
    )(hs2d, w_row, thr3d, w_bf)

    output = out2d.reshape(_B, _S, _H)
    return output, zl[0, 0], sel1d.reshape(_B, _K)


# speedup vs baseline: 1.1107x; 1.1107x over previous
import jax
import jax.numpy as jnp
from jax import lax
from jax.experimental import pallas as pl
from jax.experimental.pallas import tpu as pltpu
from jax.experimental.pallas import tpu_sc as plsc

_B, _S, _H = 4, 8192, 768
_K = _S // 2
_ALPHA = 0.1
_BS = 2048
_NBLK = (_B * _S) // _BS
_RPB = _S // _BS


def _router_body(x_ref, w_ref, lg_ref, sc_ref):
    x = x_ref[...]
    w = w_ref[...]
    lg = lax.dot_general(w, x, (((1,), (1,)), ((), ())),
                         preferred_element_type=jnp.float32)
    lg_ref[...] = lg[:, None, :]
    sc_ref[...] = _ALPHA / (1.0 + jnp.exp(-lg[:, None, :]))


def _threshold_body(sc_ref, lg_ref, thr_ref, ntk_ref, zl_ref):
    s = sc_ref[...]
    l = lg_ref[...]
    chunks = [s[b * _RPB:(b + 1) * _RPB] for b in range(_B)]

    def body(_, carry):
        los, his = carry
        new_lo, new_hi = [], []
        for b in range(_B):
            mid = los[b] + lax.shift_right_logical(his[b] - los[b], 1)
            t = lax.bitcast_convert_type(mid, jnp.float32)
            cnt = jnp.sum((chunks[b] >= t).astype(jnp.int32))
            pred = cnt >= _K
            new_lo.append(jnp.where(pred, mid, los[b]))
            new_hi.append(jnp.where(pred, his[b], mid))
        return (tuple(new_lo), tuple(new_hi))

    lo0 = tuple(jnp.int32(0) for _ in range(_B))
    hi0 = tuple(jnp.int32(0x7F800000) for _ in range(_B))
    los, _ = lax.fori_loop(0, 31, body, (lo0, hi0))
    thrs, ntks, zs = [], [], []
    for b in range(_B):
        t = lax.bitcast_convert_type(los[b], jnp.float32)
        cnt_gt = jnp.sum((chunks[b] > t).astype(jnp.int32))
        thrs.append(jnp.full((1, 16), t, jnp.float32))
        ntks.append(jnp.full((1, 16), _K - cnt_gt, jnp.int32))
        lc = l[b * _RPB:(b + 1) * _RPB]
        mx = jnp.max(lc)
        z = jnp.log(jnp.sum(jnp.exp(lc - mx))) + mx
        zs.append(z * z)
    thr_ref[...] = jnp.concatenate(thrs, axis=0)
    ntk_ref[...] = jnp.concatenate(ntks, axis=0)
    zl_ref[...] = jnp.full((1, 1), (zs[0] + zs[1] + zs[2] + zs[3]) / float(_B),
                           jnp.float32)


def _sc_compact_body(sc_hbm, thr_hbm, ntk_hbm, sel_out,
                     s_v, idx_v, thr_v, ntk_v):
    b = lax.axis_index("s") * 2 + lax.axis_index("c")

    @pl.when(b < _B)
    def _():
        pltpu.sync_copy(sc_hbm.at[pl.ds(b * _S, _S)], s_v)
        pltpu.sync_copy(thr_hbm.at[pl.ds(b * 16, 16)], thr_v)
        pltpu.sync_copy(ntk_hbm.at[pl.ds(b * 16, 16)], ntk_v)
        thr = thr_v[...]
        ntk = ntk_v[...]
        zero_i = jnp.zeros((16,), jnp.int32)
        one_i = jnp.ones((16,), jnp.int32)

        def body(i, carry):
            eq_seen, off = carry
            sl = s_v[pl.ds(i * 16, 16)]
            gt = sl > thr
            eq = sl == thr
            eqi = jnp.where(eq, one_i, zero_i)
            csum = plsc.cumsum(eqi)
            eq_seen_v = jnp.full((16,), eq_seen, jnp.int32)
            rank = (csum - eqi) + eq_seen_v
            take = jnp.logical_and(eq, rank < ntk)
            mask = jnp.logical_or(gt, take)
            idx = lax.iota(jnp.int32, 16) + jnp.full((16,), i * 16, jnp.int32)
            plsc.store_compressed(idx_v.at[pl.ds(off, 16)], idx, mask=mask)
            nsel = jnp.sum(jnp.where(mask, one_i, zero_i))
            neq = jnp.sum(eqi)
            return (eq_seen + neq, off + nsel)

        lax.fori_loop(0, _S // 16, body,
                      (jnp.int32(0), jnp.int32(0)), unroll=False)
        pltpu.sync_copy(idx_v.at[pl.ds(0, _K)], sel_out.at[pl.ds(b * _K, _K)])


def _update_body(x_ref, wr_ref, thr_ref, w_ref, o_ref):
    x = x_ref[...]
    wr = wr_ref[...]
    lg = lax.dot_general(x, wr, (((1,), (1,)), ((), ())),
                         preferred_element_type=jnp.float32)
    sc = _ALPHA / (1.0 + jnp.exp(-lg))
    thr = thr_ref[0, 0, 0]
    m = jnp.where(sc >= thr, sc, 0.0)
    y = jnp.dot(x.astype(jnp.bfloat16), w_ref[...],
                preferred_element_type=jnp.float32)
    o_ref[...] = x + m * y


def kernel(hidden_states, router_w, block_w):
    f32 = jnp.float32
    i32 = jnp.int32
    hs2d = hidden_states.reshape(_B * _S, _H)
    w_row = router_w.reshape(1, _H)

    lg_rows, sc_rows = pl.pallas_call(
        _router_body,
        grid=(_NBLK,),
        in_specs=[
            pl.BlockSpec((_BS, _H), lambda i: (i, 0)),
            pl.BlockSpec((1, _H), lambda i: (0, 0)),
        ],
        out_specs=[
            pl.BlockSpec((1, 1, _BS), lambda i: (i, 0, 0)),
            pl.BlockSpec((1, 1, _BS), lambda i: (i, 0, 0)),
        ],
        out_shape=[
            jax.ShapeDtypeStruct((_NBLK, 1, _BS), f32),
            jax.ShapeDtypeStruct((_NBLK, 1, _BS), f32),
        ],
    )(hs2d, w_row)

    thr16, ntk16, zl = pl.pallas_call(
        _threshold_body,
        out_shape=[
            jax.ShapeDtypeStruct((_B, 16), f32),
            jax.ShapeDtypeStruct((_B, 16), i32),
            jax.ShapeDtypeStruct((1, 1), f32),
        ],
    )(sc_rows, lg_rows)

    mesh = plsc.VectorSubcoreMesh(core_axis_name="c", subcore_axis_name="s")
    sel1d = pl.kernel(
        _sc_compact_body,
        out_type=jax.ShapeDtypeStruct((_B * _K,), i32),
        mesh=mesh,
        compiler_params=pltpu.CompilerParams(needs_layout_passes=False),
        scratch_types=[
            pltpu.VMEM((_S,), f32),
            pltpu.VMEM((_K + 16,), i32),
            pltpu.VMEM((16,), f32),
            pltpu.VMEM((16,), i32),
        ],
    )(sc_rows.reshape(_B * _S), thr16.reshape(_B * 16), ntk16.reshape(_B * 16))

    thr3d = thr16.reshape(_B, 1, 16)
    w_bf = block_w.astype(jnp.bfloat16)

    out2d = pl.pallas_call(
        _update_body,
        grid=(_NBLK,),
        in_specs=[
            pl.BlockSpec((_BS, _H), lambda i: (i, 0)),
            pl.BlockSpec((1, _H), lambda i: (0, 0)),
            pl.BlockSpec((1, 1, 16), lambda i: (i // _RPB, 0, 0)),
            pl.BlockSpec((_H, _H), lambda i: (0, 0)),
        ],
        out_specs=pl.BlockSpec((_BS, _H), lambda i: (i, 0)),
        out_shape=jax.ShapeDtypeStruct((_B * _S, _H), f32),
    )(hs2d, w_row, thr3d, w_bf)

    output = out2d.reshape(_B, _S, _H)
    return output, zl[0, 0], sel1d.reshape(_B, _K)


# speedup vs baseline: 1.1146x; 1.0035x over previous
import jax
import jax.numpy as jnp
from jax import lax
from jax.experimental import pallas as pl
from jax.experimental.pallas import tpu as pltpu
from jax.experimental.pallas import tpu_sc as plsc

_B, _S, _H = 4, 8192, 768
_K = _S // 2
_ALPHA = 0.1
_BS = 4096
_NBLK = (_B * _S) // _BS
_RPB = _S // _BS
_BSU = 2048
_NBU = (_B * _S) // _BSU
_RPU = _S // _BSU


def _router_body(x_ref, w_ref, lg_ref, sc_ref):
    x = x_ref[...]
    w = w_ref[...]
    lg = lax.dot_general(w, x, (((1,), (1,)), ((), ())),
                         preferred_element_type=jnp.float32)
    lg_ref[...] = lg[:, None, :]
    sc_ref[...] = _ALPHA / (1.0 + jnp.exp(-lg[:, None, :]))


def _threshold_body(sc_ref, lg_ref, thr_ref, ntk_ref, zl_ref):
    s = sc_ref[...]
    l = lg_ref[...]
    chunks = [s[b * _RPB:(b + 1) * _RPB] for b in range(_B)]

    def body(_, carry):
        los, his = carry
        new_lo, new_hi = [], []
        for b in range(_B):
            mid = los[b] + lax.shift_right_logical(his[b] - los[b], 1)
            t = lax.bitcast_convert_type(mid, jnp.float32)
            cnt = jnp.sum((chunks[b] >= t).astype(jnp.int32))
            pred = cnt >= _K
            new_lo.append(jnp.where(pred, mid, los[b]))
            new_hi.append(jnp.where(pred, his[b], mid))
        return (tuple(new_lo), tuple(new_hi))

    lo0 = tuple(jnp.int32(0) for _ in range(_B))
    hi0 = tuple(jnp.int32(0x7F800000) for _ in range(_B))
    los, _ = lax.fori_loop(0, 31, body, (lo0, hi0))
    thrs, ntks, zs = [], [], []
    for b in range(_B):
        t = lax.bitcast_convert_type(los[b], jnp.float32)
        cnt_gt = jnp.sum((chunks[b] > t).astype(jnp.int32))
        thrs.append(jnp.full((1, 16), t, jnp.float32))
        ntks.append(jnp.full((1, 16), _K - cnt_gt, jnp.int32))
        lc = l[b * _RPB:(b + 1) * _RPB]
        mx = jnp.max(lc)
        z = jnp.log(jnp.sum(jnp.exp(lc - mx))) + mx
        zs.append(z * z)
    thr_ref[...] = jnp.concatenate(thrs, axis=0)
    ntk_ref[...] = jnp.concatenate(ntks, axis=0)
    zl_ref[...] = jnp.full((1, 1), (zs[0] + zs[1] + zs[2] + zs[3]) / float(_B),
                           jnp.float32)


def _sc_compact_body(sc_hbm, thr_hbm, ntk_hbm, sel_out,
                     s_v, idx_v, thr_v, ntk_v):
    b = lax.axis_index("s") * 2 + lax.axis_index("c")

    @pl.when(b < _B)
    def _():
        pltpu.sync_copy(sc_hbm.at[pl.ds(b * _S, _S)], s_v)
        pltpu.sync_copy(thr_hbm.at[pl.ds(b * 16, 16)], thr_v)
        pltpu.sync_copy(ntk_hbm.at[pl.ds(b * 16, 16)], ntk_v)
        thr = thr_v[...]
        ntk = ntk_v[...]
        zero_i = jnp.zeros((16,), jnp.int32)
        one_i = jnp.ones((16,), jnp.int32)

        def body(i, carry):
            eq_seen, off = carry
            sl = s_v[pl.ds(i * 16, 16)]
            gt = sl > thr
            eq = sl == thr
            eqi = jnp.where(eq, one_i, zero_i)
            csum = plsc.cumsum(eqi)
            eq_seen_v = jnp.full((16,), eq_seen, jnp.int32)
            rank = (csum - eqi) + eq_seen_v
            take = jnp.logical_and(eq, rank < ntk)
            mask = jnp.logical_or(gt, take)
            idx = lax.iota(jnp.int32, 16) + jnp.full((16,), i * 16, jnp.int32)
            plsc.store_compressed(idx_v.at[pl.ds(off, 16)], idx, mask=mask)
            nsel = jnp.sum(jnp.where(mask, one_i, zero_i))
            neq = jnp.sum(eqi)
            return (eq_seen + neq, off + nsel)

        lax.fori_loop(0, _S // 16, body,
                      (jnp.int32(0), jnp.int32(0)), unroll=False)
        pltpu.sync_copy(idx_v.at[pl.ds(0, _K)], sel_out.at[pl.ds(b * _K, _K)])


def _update_body(x_ref, wr_ref, thr_ref, w_ref, o_ref):
    x = x_ref[...]
    wr = wr_ref[...]
    lg = lax.dot_general(x, wr, (((1,), (1,)), ((), ())),
                         preferred_element_type=jnp.float32)
    sc = _ALPHA / (1.0 + jnp.exp(-lg))
    thr = thr_ref[0, 0, 0]
    m = jnp.where(sc >= thr, sc, 0.0)
    y = jnp.dot(x.astype(jnp.bfloat16), w_ref[...],
                preferred_element_type=jnp.float32)
    o_ref[...] = x + m * y


def kernel(hidden_states, router_w, block_w):
    f32 = jnp.float32
    i32 = jnp.int32
    hs2d = hidden_states.reshape(_B * _S, _H)
    w_row = router_w.reshape(1, _H)

    lg_rows, sc_rows = pl.pallas_call(
        _router_body,
        grid=(_NBLK,),
        in_specs=[
            pl.BlockSpec((_BS, _H), lambda i: (i, 0)),
            pl.BlockSpec((1, _H), lambda i: (0, 0)),
        ],
        out_specs=[
            pl.BlockSpec((1, 1, _BS), lambda i: (i, 0, 0)),
            pl.BlockSpec((1, 1, _BS), lambda i: (i, 0, 0)),
        ],
        out_shape=[
            jax.ShapeDtypeStruct((_NBLK, 1, _BS), f32),
            jax.ShapeDtypeStruct((_NBLK, 1, _BS), f32),
        ],
    )(hs2d, w_row)

    thr16, ntk16, zl = pl.pallas_call(
        _threshold_body,
        out_shape=[
            jax.ShapeDtypeStruct((_B, 16), f32),
            jax.ShapeDtypeStruct((_B, 16), i32),
            jax.ShapeDtypeStruct((1, 1), f32),
        ],
    )(sc_rows, lg_rows)

    mesh = plsc.VectorSubcoreMesh(core_axis_name="c", subcore_axis_name="s")
    sel1d = pl.kernel(
        _sc_compact_body,
        out_type=jax.ShapeDtypeStruct((_B * _K,), i32),
        mesh=mesh,
        compiler_params=pltpu.CompilerParams(needs_layout_passes=False),
        scratch_types=[
            pltpu.VMEM((_S,), f32),
            pltpu.VMEM((_K + 16,), i32),
            pltpu.VMEM((16,), f32),
            pltpu.VMEM((16,), i32),
        ],
    )(sc_rows.reshape(_B * _S), thr16.reshape(_B * 16), ntk16.reshape(_B * 16))

    thr3d = thr16.reshape(_B, 1, 16)
    w_bf = block_w.astype(jnp.bfloat16)

    out2d = pl.pallas_call(
        _update_body,
        grid=(_NBU,),
        in_specs=[
            pl.BlockSpec((_BSU, _H), lambda i: (i, 0)),
            pl.BlockSpec((1, _H), lambda i: (0, 0)),
            pl.BlockSpec((1, 1, 16), lambda i: (i // _RPU, 0, 0)),
            pl.BlockSpec((_H, _H), lambda i: (0, 0)),
        ],
        out_specs=pl.BlockSpec((_BSU, _H), lambda i: (i, 0)),
        out_shape=jax.ShapeDtypeStruct((_B * _S, _H), f32),
    )(hs2d, w_row, thr3d, w_bf)

    output = out2d.reshape(_B, _S, _H)
    return output, zl[0, 0], sel1d.reshape(_B, _K)


# speedup vs baseline: 1.1301x; 1.0139x over previous
import jax
import jax.numpy as jnp
from jax import lax
from jax.experimental import pallas as pl
from jax.experimental.pallas import tpu as pltpu
from jax.experimental.pallas import tpu_sc as plsc

_B, _S, _H = 4, 8192, 768
_K = _S // 2
_ALPHA = 0.1
_BS = 4096
_NBLK = (_B * _S) // _BS
_RPB = _S // _BS
_BSU = 2048
_NBU = (_B * _S) // _BSU
_RPU = _S // _BSU


def _router_threshold_body(x_ref, w_ref, sc_ref, thr_ref, ntk_ref, zl_ref,
                           sc_acc, lg_acc):
    i = pl.program_id(0)
    x = x_ref[...]
    w = w_ref[...]
    lg = lax.dot_general(w, x, (((1,), (1,)), ((), ())),
                         preferred_element_type=jnp.float32)
    sc = _ALPHA / (1.0 + jnp.exp(-lg))
    sc_ref[...] = sc[:, None, :]
    sc_acc[pl.ds(i, 1)] = sc[:, None, :]
    lg_acc[pl.ds(i, 1)] = lg[:, None, :]

    @pl.when(i == _NBLK - 1)
    def _():
        s = sc_acc[...]
        l = lg_acc[...]
        chunks = [s[b * _RPB:(b + 1) * _RPB] for b in range(_B)]

        def body(_, carry):
            los, his = carry
            new_lo, new_hi = [], []
            for b in range(_B):
                mid = los[b] + lax.shift_right_logical(his[b] - los[b], 1)
                t = lax.bitcast_convert_type(mid, jnp.float32)
                cnt = jnp.sum((chunks[b] >= t).astype(jnp.int32))
                pred = cnt >= _K
                new_lo.append(jnp.where(pred, mid, los[b]))
                new_hi.append(jnp.where(pred, his[b], mid))
            return (tuple(new_lo), tuple(new_hi))

        lo0 = tuple(jnp.int32(0) for _ in range(_B))
        hi0 = tuple(jnp.int32(0x7F800000) for _ in range(_B))
        los, _ = lax.fori_loop(0, 31, body, (lo0, hi0))
        thrs, ntks, zs = [], [], []
        for b in range(_B):
            t = lax.bitcast_convert_type(los[b], jnp.float32)
            cnt_gt = jnp.sum((chunks[b] > t).astype(jnp.int32))
            thrs.append(jnp.full((1, 16), t, jnp.float32))
            ntks.append(jnp.full((1, 16), _K - cnt_gt, jnp.int32))
            lc = l[b * _RPB:(b + 1) * _RPB]
            mx = jnp.max(lc)
            z = jnp.log(jnp.sum(jnp.exp(lc - mx))) + mx
            zs.append(z * z)
        thr_ref[...] = jnp.concatenate(thrs, axis=0)
        ntk_ref[...] = jnp.concatenate(ntks, axis=0)
        zl_ref[...] = jnp.full((1, 1),
                               (zs[0] + zs[1] + zs[2] + zs[3]) / float(_B),
                               jnp.float32)


def _sc_compact_body(sc_hbm, thr_hbm, ntk_hbm, sel_out,
                     s_v, idx_v, thr_v, ntk_v):
    b = lax.axis_index("s") * 2 + lax.axis_index("c")

    @pl.when(b < _B)
    def _():
        pltpu.sync_copy(sc_hbm.at[pl.ds(b * _S, _S)], s_v)
        pltpu.sync_copy(thr_hbm.at[pl.ds(b * 16, 16)], thr_v)
        pltpu.sync_copy(ntk_hbm.at[pl.ds(b * 16, 16)], ntk_v)
        thr = thr_v[...]
        ntk = ntk_v[...]
        zero_i = jnp.zeros((16,), jnp.int32)
        one_i = jnp.ones((16,), jnp.int32)

        def body(i, carry):
            eq_seen, off = carry
            sl = s_v[pl.ds(i * 16, 16)]
            gt = sl > thr
            eq = sl == thr
            eqi = jnp.where(eq, one_i, zero_i)
            csum = plsc.cumsum(eqi)
            eq_seen_v = jnp.full((16,), eq_seen, jnp.int32)
            rank = (csum - eqi) + eq_seen_v
            take = jnp.logical_and(eq, rank < ntk)
            mask = jnp.logical_or(gt, take)
            idx = lax.iota(jnp.int32, 16) + jnp.full((16,), i * 16, jnp.int32)
            plsc.store_compressed(idx_v.at[pl.ds(off, 16)], idx, mask=mask)
            nsel = jnp.sum(jnp.where(mask, one_i, zero_i))
            neq = jnp.sum(eqi)
            return (eq_seen + neq, off + nsel)

        lax.fori_loop(0, _S // 16, body,
                      (jnp.int32(0), jnp.int32(0)), unroll=False)
        pltpu.sync_copy(idx_v.at[pl.ds(0, _K)], sel_out.at[pl.ds(b * _K, _K)])


def _update_body(x_ref, wr_ref, thr_ref, w_ref, o_ref):
    x = x_ref[...]
    wr = wr_ref[...]
    lg = lax.dot_general(x, wr, (((1,), (1,)), ((), ())),
                         preferred_element_type=jnp.float32)
    sc = _ALPHA / (1.0 + jnp.exp(-lg))
    thr = thr_ref[0, 0, 0]
    m = jnp.where(sc >= thr, sc, 0.0)
    y = jnp.dot(x.astype(jnp.bfloat16), w_ref[...],
                preferred_element_type=jnp.float32)
    o_ref[...] = x + m * y


def kernel(hidden_states, router_w, block_w):
    f32 = jnp.float32
    i32 = jnp.int32
    hs2d = hidden_states.reshape(_B * _S, _H)
    w_row = router_w.reshape(1, _H)

    sc_rows, thr16, ntk16, zl = pl.pallas_call(
        _router_threshold_body,
        grid=(_NBLK,),
        in_specs=[
            pl.BlockSpec((_BS, _H), lambda i: (i, 0)),
            pl.BlockSpec((1, _H), lambda i: (0, 0)),
        ],
        out_specs=[
            pl.BlockSpec((1, 1, _BS), lambda i: (i, 0, 0)),
            pl.BlockSpec((_B, 16), lambda i: (0, 0)),
            pl.BlockSpec((_B, 16), lambda i: (0, 0)),
            pl.BlockSpec((1, 1), lambda i: (0, 0)),
        ],
        out_shape=[
            jax.ShapeDtypeStruct((_NBLK, 1, _BS), f32),
            jax.ShapeDtypeStruct((_B, 16), f32),
            jax.ShapeDtypeStruct((_B, 16), i32),
            jax.ShapeDtypeStruct((1, 1), f32),
        ],
        scratch_shapes=[
            pltpu.VMEM((_NBLK, 1, _BS), jnp.float32),
            pltpu.VMEM((_NBLK, 1, _BS), jnp.float32),
        ],
    )(hs2d, w_row)

    mesh = plsc.VectorSubcoreMesh(core_axis_name="c", subcore_axis_name="s")
    sel1d = pl.kernel(
        _sc_compact_body,
        out_type=jax.ShapeDtypeStruct((_B * _K,), i32),
        mesh=mesh,
        compiler_params=pltpu.CompilerParams(needs_layout_passes=False),
        scratch_types=[
            pltpu.VMEM((_S,), f32),
            pltpu.VMEM((_K + 16,), i32),
            pltpu.VMEM((16,), f32),
            pltpu.VMEM((16,), i32),
        ],
    )(sc_rows.reshape(_B * _S), thr16.reshape(_B * 16), ntk16.reshape(_B * 16))

    thr3d = thr16.reshape(_B, 1, 16)
    w_bf = block_w.astype(jnp.bfloat16)

    out2d = pl.pallas_call(
        _update_body,
        grid=(_NBU,),
        in_specs=[
            pl.BlockSpec((_BSU, _H), lambda i: (i, 0)),
            pl.BlockSpec((1, _H), lambda i: (0, 0)),
            pl.BlockSpec((1, 1, 16), lambda i: (i // _RPU, 0, 0)),
            pl.BlockSpec((_H, _H), lambda i: (0, 0)),
        ],
        out_specs=pl.BlockSpec((_BSU, _H), lambda i: (i, 0)),
        out_shape=jax.ShapeDtypeStruct((_B * _S, _H), f32),
    )(hs2d, w_row, thr3d, w_bf)

    output = out2d.reshape(_B, _S, _H)
    return output, zl[0, 0], sel1d.reshape(_B, _K)
